# baseline (speedup 1.0000x reference)
import functools

import jax
import jax.numpy as jnp
from jax import lax
from jax.experimental import pallas as pl
from jax.experimental.pallas import tpu as pltpu
from jax.experimental.pallas import tpu_sc as plsc

N = 1048576
NT = 32
E = N // NT
L = 16
NB = 32768
NBT = NB // NT
WIN1 = 2048
WIN5 = 16384
WINCAP = WIN5 + 32
WBUF = WIN5 + 128
GPAD = N + WBUF + 128
EPS = 1e-7

_mesh = plsc.VectorSubcoreMesh(core_axis_name="c", subcore_axis_name="s")


def _wid():
  return lax.axis_index("s") * 2 + lax.axis_index("c")


def _iota16():
  return lax.iota(jnp.int32, L)


def _bcast(v, l):
  return jnp.take_along_axis(v, jnp.full((L,), l, jnp.int32), axis=0)


def _sload(ref, i):
  return ref[0, pl.ds(i, L)][0]


@functools.partial(
    pl.kernel,
    out_type=jax.ShapeDtypeStruct((NT, NB), jnp.int32),
    mesh=_mesh,
    compiler_params=pltpu.CompilerParams(needs_layout_passes=False),
    scratch_types=[
        pltpu.VMEM((WIN1,), jnp.float32),
        pltpu.VMEM((NB,), jnp.int32),
    ],
)
def _p1(t_hbm, cnt_out, twin, hist):
  wid = _wid()

  def zero(i, _):
    hist[pl.ds(i * L, L)] = jnp.zeros((L,), jnp.int32)
    return 0

  lax.fori_loop(0, NB // L, zero, 0)

  def win(wi, _):
    base = wid * E + wi * WIN1
    pltpu.sync_copy(t_hbm.at[pl.ds(base, WIN1)], twin)

    def vreg(vi, _):
      t = twin[pl.ds(vi * L, L)]
      b = (t * jnp.float32(NB)).astype(jnp.int32)
      occ, last = plsc.scan_count(b)
      cur = plsc.load_gather(hist, [b])
      plsc.store_scatter(hist, [b], cur + occ, mask=last)
      return 0

    lax.fori_loop(0, WIN1 // L, vreg, 0)
    return 0

  lax.fori_loop(0, E // WIN1, win, 0)
  pltpu.sync_copy(hist, cnt_out.at[wid])


def _cs(x, axis):
  n = x.shape[axis]
  s = 1
  while s < n:
    shifted = lax.slice_in_dim(x, 0, n - s, axis=axis)
    if axis == 0:
      pad = jnp.zeros((s,) + x.shape[1:], x.dtype)
    else:
      pad = jnp.zeros(x.shape[:1] + (s,), x.dtype)
    x = x + jnp.concatenate([pad, shifted], axis=axis)
    s *= 2
  return x


def _p2_body(cnt_ref, base_ref, seg_ref):
  cnt = cnt_ref[...].astype(jnp.float32)
  chunkpre = _cs(cnt, 0) - cnt
  cc = jnp.sum(cnt, axis=0)
  R, C = NB // 128, 128
  y = cc.reshape(R, C)
  rowincl = _cs(y, 1)
  rowsum = rowincl[:, C - 1:C]
  rowoff = _cs(rowsum, 0) - rowsum
  gs = (rowincl - y + rowoff).reshape(1, NB)
  base_ref[...] = (gs + chunkpre).astype(jnp.int32)
  seg_ref[...] = jnp.concatenate(
      [gs.astype(jnp.int32), jnp.full((1, 128), N, jnp.int32)], axis=1)


def _p2(cnths):
  return pl.pallas_call(
      _p2_body,
      out_shape=[
          jax.ShapeDtypeStruct((NT, NB), jnp.int32),
          jax.ShapeDtypeStruct((1, NB + 128), jnp.int32),
      ],
  )(cnths)


@functools.partial(
    pl.kernel,
    out_type=[
        jax.ShapeDtypeStruct((GPAD,), jnp.float32),
        jax.ShapeDtypeStruct((GPAD,), jnp.float32),
        jax.ShapeDtypeStruct((GPAD,), jnp.int32),
    ],
    mesh=_mesh,
    compiler_params=pltpu.CompilerParams(needs_layout_passes=False),
    scratch_types=[
        pltpu.VMEM((WIN1,), jnp.float32),
        pltpu.VMEM((WIN1,), jnp.float32),
        pltpu.VMEM((NB,), jnp.int32),
        pltpu.VMEM((WIN1 // 128, 128), jnp.int32),
        pltpu.VMEM((WIN1 // 128, 128), jnp.float32),
        pltpu.VMEM((WIN1 // 128, 128), jnp.float32),
        pltpu.VMEM((WIN1 // 128, 128), jnp.int32),
        pltpu.VMEM((3 * WIN1,), jnp.float32),
        pltpu.SemaphoreType.DMA,
    ],
)
def _p3(t_hbm, p_hbm, base_hbm, gt, gw, gi,
        twin, pwin, cur, posb, tb, wb, ib, dumv, sem):
  wid = _wid()
  pltpu.sync_copy(base_hbm.at[wid], cur)
  nrow = WIN1 // 128

  def win(wi, _):
    off = wid * E + wi * WIN1
    pltpu.sync_copy(t_hbm.at[pl.ds(off, WIN1)], twin)
    pltpu.sync_copy(p_hbm.at[pl.ds(off, WIN1)], pwin)

    def vreg(vi, _):
      t = twin[pl.ds(vi * L, L)]
      p = pwin[pl.ds(vi * L, L)]
      b = (t * jnp.float32(NB)).astype(jnp.int32)
      occ, last = plsc.scan_count(b)
      c0 = plsc.load_gather(cur, [b])
      pos = c0 + occ - 1
      plsc.store_scatter(cur, [b], pos + 1, mask=last)
      row = vi // 8
      col = (vi % 8) * L
      posb[row, pl.ds(col, L)] = pos
      tb[row, pl.ds(col, L)] = t
      wb[row, pl.ds(col, L)] = jnp.exp(p)
      ib[row, pl.ds(col, L)] = off + vi * L + _iota16()
      return 0

    lax.fori_loop(0, WIN1 // L, vreg, 0)

    def flush(r, _):
      idx = posb.at[r]
      pltpu.async_copy(tb.at[r], gt.at[idx], sem)
      pltpu.async_copy(wb.at[r], gw.at[idx], sem)
      pltpu.async_copy(ib.at[r], gi.at[idx], sem)
      return 0

    lax.fori_loop(0, nrow, flush, 0)
    pltpu.make_async_copy(t_hbm.at[pl.ds(0, 3 * WIN1)], dumv, sem).wait()
    return 0

  lax.fori_loop(0, E // WIN1, win, 0)


@functools.partial(
    pl.kernel,
    out_type=jax.ShapeDtypeStruct((NT, L), jnp.float32),
    mesh=_mesh,
    compiler_params=pltpu.CompilerParams(needs_layout_passes=False),
    scratch_types=[
        pltpu.VMEM((WIN1,), jnp.float32),
        pltpu.VMEM((L,), jnp.float32),
        pltpu.VMEM((1, 128), jnp.int32),
        pltpu.VMEM((1, 128), jnp.int32),
    ],
)
def _p4(gw_hbm, seg_hbm, ts_out, wwin, accv, sma, smb):
  wid = _wid()
  pltpu.sync_copy(seg_hbm.at[pl.ds(0, 1), pl.ds(wid * NBT, 128)], sma)
  pltpu.sync_copy(seg_hbm.at[pl.ds(0, 1), pl.ds((wid + 1) * NBT, 128)], smb)
  lo = sma[0, pl.ds(0, L)][0]
  hi = smb[0, pl.ds(0, L)][0]
  start = pl.multiple_of(lax.bitwise_and(lo, jnp.int32(-16)), 16)
  nwin = (hi - start + WIN1 - 1) // WIN1

  def win(wi, acc):
    pltpu.sync_copy(gw_hbm.at[pl.ds(start + wi * WIN1, WIN1)], wwin)

    def vreg(vi, acc):
      w = wwin[pl.ds(vi * L, L)]
      pos = start + wi * WIN1 + vi * L + _iota16()
      ok = (pos >= lo) & (pos < hi)
      return acc + jnp.where(ok, w, jnp.float32(0.0))

    return lax.fori_loop(0, WIN1 // L, vreg, acc)

  acc = lax.fori_loop(0, nwin, win, jnp.zeros((L,), jnp.float32))
  accv[...] = acc
  pltpu.sync_copy(accv, ts_out.at[wid])


@functools.partial(
    pl.kernel,
    out_type=jax.ShapeDtypeStruct((GPAD,), jnp.float32),
    mesh=_mesh,
    compiler_params=pltpu.CompilerParams(needs_layout_passes=False),
    scratch_types=[
        pltpu.VMEM((WBUF,), jnp.float32),
        pltpu.VMEM((WBUF,), jnp.float32),
        pltpu.VMEM((WBUF,), jnp.int32),
        pltpu.VMEM((WBUF // 128 + 1, 128), jnp.float32),
        pltpu.VMEM((WBUF // 128 + 1, 128), jnp.int32),
        pltpu.VMEM((WBUF + 128,), jnp.float32),
        pltpu.VMEM((NT, L), jnp.float32),
        pltpu.VMEM((1, NBT + 128), jnp.int32),
        pltpu.SemaphoreType.DMA,
    ],
)
def _p5(gt_hbm, gw_hbm, gi_hbm, seg_hbm, ts_hbm, s_out,
        tw, ww, iw, sval, sidx, dumv, tsv, seg, sem):
  wid = _wid()
  pltpu.sync_copy(
      seg_hbm.at[pl.ds(0, 1), pl.ds(wid * NBT, NBT + 128)], seg)
  pltpu.sync_copy(ts_hbm, tsv)

  def addrow(i, a):
    return a + tsv[i]

  carry = jnp.sum(lax.fori_loop(0, wid, addrow, jnp.zeros((L,), jnp.float32)))
  wtot = jnp.sum(lax.fori_loop(0, NT, addrow, jnp.zeros((L,), jnp.float32)))
  nrow = WBUF // 128 + 1
  iota = _iota16()

  def outer_cond(st):
    kk, _ = st
    return kk < NBT

  def outer_body(st):
    kk_entry, r0 = st
    s0 = _sload(seg, kk_entry)
    ws = pl.multiple_of(lax.bitwise_and(s0, jnp.int32(-16)), 16)
    pltpu.sync_copy(gt_hbm.at[pl.ds(ws, WBUF)], tw)
    pltpu.sync_copy(gw_hbm.at[pl.ds(ws, WBUF)], ww)
    pltpu.sync_copy(gi_hbm.at[pl.ds(ws, WBUF)], iw)

    def dmp(q, _):
      row = q // 8
      col = (q % 8) * L
      flat = row * 128 + col + iota
      sidx[row, pl.ds(col, L)] = N + 16 + lax.bitwise_and(flat, 16383)
      return 0

    lax.fori_loop(0, nrow * 8, dmp, 0)

    def inner_cond(st2):
      kk, _ = st2
      return (kk < NBT) & (
          (kk == kk_entry) | (_sload(seg, kk + 1) - ws <= WINCAP))

    def inner_body(st2):
      kk, r = st2
      s = _sload(seg, kk)
      e = _sload(seg, kk + 1)
      o = s - ws
      kcnt = jnp.minimum(e - s, WINCAP - o)
      nj = (kcnt + L - 1) // L
      oa = lax.bitwise_and(o, jnp.int32(-16))
      ni = (o - oa + kcnt + L - 1) // L

      def jsum(jj, a):
        w = ww[pl.ds(o + jj * L, L)]
        ok = (jj * L + iota) < kcnt
        return a + jnp.where(ok, w, jnp.float32(0.0))

      sb = jnp.sum(lax.fori_loop(0, nj, jsum, jnp.zeros((L,), jnp.float32)))
      sbase = wtot - (r + sb)

      def iloop(ii, _):
        io = oa + ii * L
        ti = tw[pl.ds(io, L)]
        ai = iw[pl.ds(io, L)]
        posi = io + iota
        vi = (posi >= o) & (posi < o + kcnt)

        def jloop(jj, acc):
          jo = o + jj * L
          tj = tw[pl.ds(jo, L)]
          aj = iw[pl.ds(jo, L)]
          wj = ww[pl.ds(jo, L)]
          wjv = jnp.where((jj * L + iota) < kcnt, wj, jnp.float32(0.0))
          for l in range(L):
            tb = _bcast(tj, l)
            ab = _bcast(aj, l)
            wb = _bcast(wjv, l)
            m = (tb > ti) | ((tb == ti) & (ab <= ai))
            acc = acc + jnp.where(m, wb, jnp.float32(0.0))
          return acc

        acc = lax.fori_loop(0, nj, jloop, jnp.zeros((L,), jnp.float32))
        row = io // 128
        col = io - row * 128
        oldv = sval[row, pl.ds(col, L)]
        oldi = sidx[row, pl.ds(col, L)]
        sval[row, pl.ds(col, L)] = jnp.where(vi, sbase + acc, oldv)
        sidx[row, pl.ds(col, L)] = jnp.where(vi, ai, oldi)
        return 0

      lax.fori_loop(0, ni, iloop, 0)
      return kk + 1, r + sb

    kk2, r2 = lax.while_loop(inner_cond, inner_body, (kk_entry, r0))

    def flush(q, _):
      pltpu.async_copy(sval.at[q], s_out.at[sidx.at[q]], sem)
      return 0

    lax.fori_loop(0, nrow, flush, 0)
    pltpu.make_async_copy(
        gt_hbm.at[pl.ds(0, nrow * 128)], dumv, sem).wait()
    return kk2, r2

  lax.while_loop(outer_cond, outer_body, (jnp.int32(0), carry))


def _p6_body(p_ref, c_ref, s_ref, nl_ref, ev_ref, loss_ref):
  i = pl.program_id(0)
  p = p_ref[...]
  c = c_ref[...]
  s = s_ref[...]
  contrib = c * (p - jnp.log(s + jnp.float32(EPS)))
  psum = jnp.sum(contrib)
  esum = jnp.sum(c)

  @pl.when(i == 0)
  def _():
    nl_ref[...] = psum.reshape(1, 1)
    ev_ref[...] = esum.reshape(1, 1)

  @pl.when(i != 0)
  def _():
    nl_ref[...] += psum.reshape(1, 1)
    ev_ref[...] += esum.reshape(1, 1)

  @pl.when(i == pl.num_programs(0) - 1)
  def _():
    loss_ref[...] = -nl_ref[...] / (ev_ref[...] + jnp.float32(EPS))


def _p6(p2d, c2d, s2d):
  g = 8
  rows = p2d.shape[0] // g
  bs = pl.BlockSpec((rows, p2d.shape[1]), lambda i: (i, 0))
  os = pl.BlockSpec((1, 1), lambda i: (0, 0))
  return pl.pallas_call(
      _p6_body,
      grid=(g,),
      in_specs=[bs, bs, bs],
      out_specs=[os, os, os],
      out_shape=[jax.ShapeDtypeStruct((1, 1), jnp.float32)] * 3,
  )(p2d, c2d, s2d)


def kernel(predictions, event_times, censored):
  cnths = _p1(event_times)
  base, seg = _p2(cnths)
  gt, gw, gi = _p3(event_times, predictions, base)
  ts = _p4(gw, seg)
  s_full = _p5(gt, gw, gi, seg, ts)
  shape2 = (512, 2048)
  _, _, loss = _p6(
      predictions.reshape(shape2),
      censored.reshape(shape2),
      s_full[:N].reshape(shape2),
  )
  return loss[0, 0]


# speedup vs baseline: 1.0001x; 1.0001x over previous
import functools

import jax
import jax.numpy as jnp
from jax import lax
from jax.experimental import pallas as pl
from jax.experimental.pallas import tpu as pltpu
from jax.experimental.pallas import tpu_sc as plsc

N = 1048576
NT = 32
E = N // NT
L = 16
NB = 32768
NBT = NB // NT
WIN1 = 2048
WIN5 = 16384
WINCAP = WIN5 + 32
WBUF = WIN5 + 128
GPAD = N + WBUF + 128
EPS = 1e-7

_mesh = plsc.VectorSubcoreMesh(core_axis_name="c", subcore_axis_name="s")


def _wid():
  return lax.axis_index("s") * 2 + lax.axis_index("c")


def _iota16():
  return lax.iota(jnp.int32, L)


def _bcast(v, l):
  return jnp.take_along_axis(v, jnp.full((L,), l, jnp.int32), axis=0)


def _sload(ref, i):
  return ref[0, pl.ds(i, L)][0]


@functools.partial(
    pl.kernel,
    out_type=jax.ShapeDtypeStruct((NT, NB), jnp.int32),
    mesh=_mesh,
    compiler_params=pltpu.CompilerParams(needs_layout_passes=False),
    scratch_types=[
        pltpu.VMEM((WIN1,), jnp.float32),
        pltpu.VMEM((NB,), jnp.int32),
    ],
)
def _p1(t_hbm, cnt_out, twin, hist):
  wid = _wid()

  def zero(i, _):
    hist[pl.ds(i * L, L)] = jnp.zeros((L,), jnp.int32)
    return 0

  lax.fori_loop(0, NB // L, zero, 0)

  def win(wi, _):
    base = wid * E + wi * WIN1
    pltpu.sync_copy(t_hbm.at[pl.ds(base, WIN1)], twin)

    def vreg(vi, _):
      t = twin[pl.ds(vi * L, L)]
      b = (t * jnp.float32(NB)).astype(jnp.int32)
      occ, last = plsc.scan_count(b)
      cur = plsc.load_gather(hist, [b])
      plsc.store_scatter(hist, [b], cur + occ, mask=last)
      return 0

    lax.fori_loop(0, WIN1 // L, vreg, 0)
    return 0

  lax.fori_loop(0, E // WIN1, win, 0)
  pltpu.sync_copy(hist, cnt_out.at[wid])


def _cs(x, axis):
  n = x.shape[axis]
  s = 1
  while s < n:
    shifted = lax.slice_in_dim(x, 0, n - s, axis=axis)
    if axis == 0:
      pad = jnp.zeros((s,) + x.shape[1:], x.dtype)
    else:
      pad = jnp.zeros(x.shape[:1] + (s,), x.dtype)
    x = x + jnp.concatenate([pad, shifted], axis=axis)
    s *= 2
  return x


def _p2_body(cnt_ref, base_ref, seg_ref):
  cnt = cnt_ref[...].astype(jnp.float32)
  chunkpre = _cs(cnt, 0) - cnt
  cc = jnp.sum(cnt, axis=0)
  R, C = NB // 128, 128
  y = cc.reshape(R, C)
  rowincl = _cs(y, 1)
  rowsum = rowincl[:, C - 1:C]
  rowoff = _cs(rowsum, 0) - rowsum
  gs = (rowincl - y + rowoff).reshape(1, NB)
  base_ref[...] = (gs + chunkpre).astype(jnp.int32)
  seg_ref[...] = jnp.concatenate(
      [gs.astype(jnp.int32), jnp.full((1, 128), N, jnp.int32)], axis=1)


def _p2(cnths):
  return pl.pallas_call(
      _p2_body,
      out_shape=[
          jax.ShapeDtypeStruct((NT, NB), jnp.int32),
          jax.ShapeDtypeStruct((1, NB + 128), jnp.int32),
      ],
  )(cnths)


@functools.partial(
    pl.kernel,
    out_type=[
        jax.ShapeDtypeStruct((GPAD,), jnp.float32),
        jax.ShapeDtypeStruct((GPAD,), jnp.float32),
        jax.ShapeDtypeStruct((GPAD,), jnp.int32),
    ],
    mesh=_mesh,
    compiler_params=pltpu.CompilerParams(needs_layout_passes=False),
    scratch_types=[
        pltpu.VMEM((WIN1,), jnp.float32),
        pltpu.VMEM((WIN1,), jnp.float32),
        pltpu.VMEM((NB,), jnp.int32),
        pltpu.VMEM((WIN1,), jnp.int32),
        pltpu.VMEM((WIN1,), jnp.float32),
        pltpu.VMEM((WIN1,), jnp.float32),
        pltpu.VMEM((WIN1,), jnp.int32),
        pltpu.VMEM((3 * WIN1,), jnp.float32),
        pltpu.SemaphoreType.DMA,
    ],
)
def _p3(t_hbm, p_hbm, base_hbm, gt, gw, gi,
        twin, pwin, cur, posb, tb, wb, ib, dumv, sem):
  wid = _wid()
  pltpu.sync_copy(base_hbm.at[wid], cur)
  nrow = WIN1 // 128

  def win(wi, _):
    off = wid * E + wi * WIN1
    pltpu.sync_copy(t_hbm.at[pl.ds(off, WIN1)], twin)
    pltpu.sync_copy(p_hbm.at[pl.ds(off, WIN1)], pwin)

    def vreg(vi, _):
      t = twin[pl.ds(vi * L, L)]
      p = pwin[pl.ds(vi * L, L)]
      b = (t * jnp.float32(NB)).astype(jnp.int32)
      occ, last = plsc.scan_count(b)
      c0 = plsc.load_gather(cur, [b])
      pos = c0 + occ - 1
      plsc.store_scatter(cur, [b], pos + 1, mask=last)
      posb[pl.ds(vi * L, L)] = pos
      tb[pl.ds(vi * L, L)] = t
      wb[pl.ds(vi * L, L)] = jnp.exp(p)
      ib[pl.ds(vi * L, L)] = off + vi * L + _iota16()
      return 0

    lax.fori_loop(0, WIN1 // L, vreg, 0)

    pltpu.async_copy(tb, gt.at[posb], sem)
    pltpu.async_copy(wb, gw.at[posb], sem)
    pltpu.async_copy(ib, gi.at[posb], sem)
    pltpu.make_async_copy(t_hbm.at[pl.ds(0, 3 * WIN1)], dumv, sem).wait()
    return 0

  lax.fori_loop(0, E // WIN1, win, 0)


@functools.partial(
    pl.kernel,
    out_type=jax.ShapeDtypeStruct((NT, L), jnp.float32),
    mesh=_mesh,
    compiler_params=pltpu.CompilerParams(needs_layout_passes=False),
    scratch_types=[
        pltpu.VMEM((WIN1,), jnp.float32),
        pltpu.VMEM((L,), jnp.float32),
        pltpu.VMEM((1, 128), jnp.int32),
        pltpu.VMEM((1, 128), jnp.int32),
    ],
)
def _p4(gw_hbm, seg_hbm, ts_out, wwin, accv, sma, smb):
  wid = _wid()
  pltpu.sync_copy(seg_hbm.at[pl.ds(0, 1), pl.ds(wid * NBT, 128)], sma)
  pltpu.sync_copy(seg_hbm.at[pl.ds(0, 1), pl.ds((wid + 1) * NBT, 128)], smb)
  lo = sma[0, pl.ds(0, L)][0]
  hi = smb[0, pl.ds(0, L)][0]
  start = pl.multiple_of(lax.bitwise_and(lo, jnp.int32(-16)), 16)
  nwin = (hi - start + WIN1 - 1) // WIN1

  def win(wi, acc):
    pltpu.sync_copy(gw_hbm.at[pl.ds(start + wi * WIN1, WIN1)], wwin)

    def vreg(vi, acc):
      w = wwin[pl.ds(vi * L, L)]
      pos = start + wi * WIN1 + vi * L + _iota16()
      ok = (pos >= lo) & (pos < hi)
      return acc + jnp.where(ok, w, jnp.float32(0.0))

    return lax.fori_loop(0, WIN1 // L, vreg, acc)

  acc = lax.fori_loop(0, nwin, win, jnp.zeros((L,), jnp.float32))
  accv[...] = acc
  pltpu.sync_copy(accv, ts_out.at[wid])


@functools.partial(
    pl.kernel,
    out_type=jax.ShapeDtypeStruct((GPAD,), jnp.float32),
    mesh=_mesh,
    compiler_params=pltpu.CompilerParams(needs_layout_passes=False),
    scratch_types=[
        pltpu.VMEM((WBUF,), jnp.float32),
        pltpu.VMEM((WBUF,), jnp.float32),
        pltpu.VMEM((WBUF,), jnp.int32),
        pltpu.VMEM((WBUF + 128,), jnp.float32),
        pltpu.VMEM((WBUF + 128,), jnp.int32),
        pltpu.VMEM((WBUF + 128,), jnp.float32),
        pltpu.VMEM((NT, L), jnp.float32),
        pltpu.VMEM((1, NBT + 128), jnp.int32),
        pltpu.SemaphoreType.DMA,
    ],
)
def _p5(gt_hbm, gw_hbm, gi_hbm, seg_hbm, ts_hbm, s_out,
        tw, ww, iw, sval, sidx, dumv, tsv, seg, sem):
  wid = _wid()
  pltpu.sync_copy(
      seg_hbm.at[pl.ds(0, 1), pl.ds(wid * NBT, NBT + 128)], seg)
  pltpu.sync_copy(ts_hbm, tsv)

  def addrow(i, a):
    return a + tsv[i]

  carry = jnp.sum(lax.fori_loop(0, wid, addrow, jnp.zeros((L,), jnp.float32)))
  wtot = jnp.sum(lax.fori_loop(0, NT, addrow, jnp.zeros((L,), jnp.float32)))
  nrow = WBUF // 128 + 1
  iota = _iota16()

  def outer_cond(st):
    kk, _ = st
    return kk < NBT

  def outer_body(st):
    kk_entry, r0 = st
    s0 = _sload(seg, kk_entry)
    ws = pl.multiple_of(lax.bitwise_and(s0, jnp.int32(-16)), 16)
    pltpu.sync_copy(gt_hbm.at[pl.ds(ws, WBUF)], tw)
    pltpu.sync_copy(gw_hbm.at[pl.ds(ws, WBUF)], ww)
    pltpu.sync_copy(gi_hbm.at[pl.ds(ws, WBUF)], iw)

    def dmp(q, _):
      flat = q * L + iota
      sidx[pl.ds(q * L, L)] = N + 16 + lax.bitwise_and(flat, 16383)
      return 0

    lax.fori_loop(0, (WBUF + 128) // L, dmp, 0)

    def inner_cond(st2):
      kk, _ = st2
      return (kk < NBT) & (
          (kk == kk_entry) | (_sload(seg, kk + 1) - ws <= WINCAP))

    def inner_body(st2):
      kk, r = st2
      s = _sload(seg, kk)
      e = _sload(seg, kk + 1)
      o = s - ws
      kcnt = jnp.minimum(e - s, WINCAP - o)
      nj = (kcnt + L - 1) // L
      oa = lax.bitwise_and(o, jnp.int32(-16))
      ni = (o - oa + kcnt + L - 1) // L

      def jsum(jj, a):
        w = ww[pl.ds(o + jj * L, L)]
        ok = (jj * L + iota) < kcnt
        return a + jnp.where(ok, w, jnp.float32(0.0))

      sb = jnp.sum(lax.fori_loop(0, nj, jsum, jnp.zeros((L,), jnp.float32)))
      sbase = wtot - (r + sb)

      def iloop(ii, _):
        io = oa + ii * L
        ti = tw[pl.ds(io, L)]
        ai = iw[pl.ds(io, L)]
        posi = io + iota
        vi = (posi >= o) & (posi < o + kcnt)

        def jloop(jj, acc):
          jo = o + jj * L
          tj = tw[pl.ds(jo, L)]
          aj = iw[pl.ds(jo, L)]
          wj = ww[pl.ds(jo, L)]
          wjv = jnp.where((jj * L + iota) < kcnt, wj, jnp.float32(0.0))
          for l in range(L):
            tb = _bcast(tj, l)
            ab = _bcast(aj, l)
            wb = _bcast(wjv, l)
            m = (tb > ti) | ((tb == ti) & (ab <= ai))
            acc = acc + jnp.where(m, wb, jnp.float32(0.0))
          return acc

        acc = lax.fori_loop(0, nj, jloop, jnp.zeros((L,), jnp.float32))
        oldv = sval[pl.ds(io, L)]
        oldi = sidx[pl.ds(io, L)]
        sval[pl.ds(io, L)] = jnp.where(vi, sbase + acc, oldv)
        sidx[pl.ds(io, L)] = jnp.where(vi, ai, oldi)
        return 0

      lax.fori_loop(0, ni, iloop, 0)
      return kk + 1, r + sb

    kk2, r2 = lax.while_loop(inner_cond, inner_body, (kk_entry, r0))

    pltpu.async_copy(sval, s_out.at[sidx], sem)
    pltpu.make_async_copy(
        gt_hbm.at[pl.ds(0, WBUF + 128)], dumv, sem).wait()
    return kk2, r2

  lax.while_loop(outer_cond, outer_body, (jnp.int32(0), carry))


def _p6_body(p_ref, c_ref, s_ref, nl_ref, ev_ref, loss_ref):
  i = pl.program_id(0)
  p = p_ref[...]
  c = c_ref[...]
  s = s_ref[...]
  contrib = c * (p - jnp.log(s + jnp.float32(EPS)))
  psum = jnp.sum(contrib)
  esum = jnp.sum(c)

  @pl.when(i == 0)
  def _():
    nl_ref[...] = psum.reshape(1, 1)
    ev_ref[...] = esum.reshape(1, 1)

  @pl.when(i != 0)
  def _():
    nl_ref[...] += psum.reshape(1, 1)
    ev_ref[...] += esum.reshape(1, 1)

  @pl.when(i == pl.num_programs(0) - 1)
  def _():
    loss_ref[...] = -nl_ref[...] / (ev_ref[...] + jnp.float32(EPS))


def _p6(p2d, c2d, s2d):
  g = 8
  rows = p2d.shape[0] // g
  bs = pl.BlockSpec((rows, p2d.shape[1]), lambda i: (i, 0))
  os = pl.BlockSpec((1, 1), lambda i: (0, 0))
  return pl.pallas_call(
      _p6_body,
      grid=(g,),
      in_specs=[bs, bs, bs],
      out_specs=[os, os, os],
      out_shape=[jax.ShapeDtypeStruct((1, 1), jnp.float32)] * 3,
  )(p2d, c2d, s2d)


def kernel(predictions, event_times, censored):
  cnths = _p1(event_times)
  base, seg = _p2(cnths)
  gt, gw, gi = _p3(event_times, predictions, base)
  ts = _p4(gw, seg)
  s_full = _p5(gt, gw, gi, seg, ts)
  shape2 = (512, 2048)
  _, _, loss = _p6(
      predictions.reshape(shape2),
      censored.reshape(shape2),
      s_full[:N].reshape(shape2),
  )
  return loss[0, 0]
